# Initial kernel scaffold; baseline (speedup 1.0000x reference)
import jax
import jax.numpy as jnp
from jax.experimental import pallas as pl


def kernel(x, edge_index, W, att_src, att_dst, bias):
    raise NotImplementedError("write your pallas kernel here")


# speedup vs baseline: 23.1116x; 23.1116x over previous
import functools

import jax
import jax.numpy as jnp
from jax import lax
from jax.experimental import pallas as pl
from jax.experimental.pallas import tpu as pltpu
from jax.experimental.pallas import tpu_sc as plsc

N = 10000
D = 128

NC = 2
NS = 16
NW = NC * NS

E_TOT = 320000 + N
CB = 128
STEPS = 81
E_PER_W = CB * STEPS
E_PAD = E_PER_W * NW
N_PAD = 10240
ROWS_T = N_PAD // NS
DEN_W = 16

BLK = 128
GRID_N = (N + BLK - 1) // BLK
GRID_NP = N_PAD // BLK


def _proj_body(x_ref, w_ref, att_ref, h_ref, a2_ref):
    h = jnp.dot(x_ref[...], w_ref[...], preferred_element_type=jnp.float32)
    h_ref[...] = h
    a2_ref[0, :] = jnp.sum(h * att_ref[0:1, :], axis=1)
    a2_ref[1, :] = jnp.sum(h * att_ref[1:2, :], axis=1)


def _project(x, W, att):
    return pl.pallas_call(
        _proj_body,
        grid=(GRID_N,),
        in_specs=[
            pl.BlockSpec((BLK, D), lambda i: (i, 0)),
            pl.BlockSpec((D, D), lambda i: (0, 0)),
            pl.BlockSpec((2, D), lambda i: (0, 0)),
        ],
        out_specs=[
            pl.BlockSpec((BLK, D), lambda i: (i, 0)),
            pl.BlockSpec((2, BLK), lambda i: (0, i)),
        ],
        out_shape=[
            jax.ShapeDtypeStruct((N, D), jnp.float32),
            jax.ShapeDtypeStruct((2, N), jnp.float32),
        ],
    )(x, W, att)


def _sc_body(src_h, dst_h, h_h, as_h, ad_h,
             acc_out, den_out,
             sidx_v, didx_v, asrc_v, adst_v, rows_v, e_v, dstage_v,
             acc_sh, den_sh, sem):
    cid = lax.axis_index("c")
    sid = lax.axis_index("s")
    wid = cid * NS + sid

    pltpu.sync_copy(as_h, asrc_v)
    pltpu.sync_copy(ad_h, adst_v)

    z16 = jnp.zeros((16,), jnp.float32)

    def _zrow(r, _):
        for c in range(D // 16):
            rows_v[r, pl.ds(c * 16, 16)] = z16
        return 0
    lax.fori_loop(0, CB, _zrow, 0)

    def _zdrow(r, _):
        dstage_v[pl.ds(r * 16, 16)] = z16
        return 0
    lax.fori_loop(0, ROWS_T // 16, _zdrow, 0)

    base = sid * ROWS_T
    for i in range(5):
        pltpu.sync_copy(rows_v, acc_sh.at[pl.ds(base + i * 128, 128)])
    pltpu.sync_copy(dstage_v, den_sh.at[pl.ds(base, ROWS_T)])
    plsc.subcore_barrier()

    iota16 = lax.iota(jnp.int32, 16)
    edge0 = wid * E_PER_W

    def _step(j, _):
        off = pl.multiple_of(edge0 + j * CB, CB)
        pltpu.sync_copy(src_h.at[pl.ds(off, CB)], sidx_v)
        pltpu.sync_copy(dst_h.at[pl.ds(off, CB)], didx_v)

        for k in range(CB // 16):
            s_idx = sidx_v[pl.ds(k * 16, 16)]
            d_idx = didx_v[pl.ds(k * 16, 16)]
            s = plsc.load_gather(asrc_v, [s_idx])
            d = plsc.load_gather(adst_v, [d_idx])
            z = s + d
            z = jnp.maximum(z, 0.0) + 0.2 * jnp.minimum(z, 0.0)
            e = jnp.exp(z)
            glob = off + k * 16 + iota16
            e = jnp.where(glob < E_TOT, e, 0.0)
            e_v[pl.ds(k * 16, 16)] = e

        pltpu.async_copy(h_h.at[sidx_v], rows_v, sem).wait()

        def _scale(g, _):
            ev = e_v[pl.ds(g * 16, 16)]
            for rr in range(16):
                er = ev[rr]
                r = g * 16 + rr
                for c in range(D // 16):
                    rows_v[r, pl.ds(c * 16, 16)] = (
                        rows_v[r, pl.ds(c * 16, 16)] * er)
            return 0
        lax.fori_loop(0, CB // 16, _scale, 0)

        pltpu.sync_copy(rows_v, acc_sh.at[didx_v], add=True)
        pltpu.sync_copy(e_v, den_sh.at[didx_v], add=True)
        return 0

    lax.fori_loop(0, STEPS, _step, 0)
    plsc.subcore_barrier()

    for i in range(5):
        pltpu.sync_copy(acc_sh.at[pl.ds(base + i * 128, 128)], rows_v)
        pltpu.sync_copy(rows_v, acc_out.at[cid, pl.ds(base + i * 128, 128)])

    pltpu.sync_copy(den_sh.at[pl.ds(base, ROWS_T)], dstage_v)
    pltpu.sync_copy(dstage_v, den_out.at[cid, pl.ds(base, ROWS_T)])


def _sc_aggregate(src3, dst3, h, a_src, a_dst):
    mesh = plsc.VectorSubcoreMesh(core_axis_name="c", subcore_axis_name="s",
                                  num_cores=NC, num_subcores=NS)
    f = pl.kernel(
        _sc_body,
        out_type=[
            jax.ShapeDtypeStruct((NC, N_PAD, D), jnp.float32),
            jax.ShapeDtypeStruct((NC, N_PAD), jnp.float32),
        ],
        mesh=mesh,
        compiler_params=pltpu.CompilerParams(needs_layout_passes=False),
        scratch_types=[
            pltpu.VMEM((CB,), jnp.int32),
            pltpu.VMEM((CB,), jnp.int32),
            pltpu.VMEM((N,), jnp.float32),
            pltpu.VMEM((N,), jnp.float32),
            pltpu.VMEM((CB, D), jnp.float32),
            pltpu.VMEM((CB,), jnp.float32),
            pltpu.VMEM((ROWS_T,), jnp.float32),
            pltpu.VMEM_SHARED((N_PAD, D), jnp.float32),
            pltpu.VMEM_SHARED((N_PAD,), jnp.float32),
            pltpu.SemaphoreType.DMA,
        ],
    )
    return f(src3, dst3, h, a_src, a_dst)


def _comb_body(acc_ref, den_ref, b_ref, o_ref):
    p = acc_ref[0] + acc_ref[1]
    dn = den_ref[0:1, :] + den_ref[1:2, :]
    dinv = jnp.eye(BLK, dtype=jnp.float32) * (1.0 / (dn + 1e-16))
    o_ref[...] = jnp.dot(dinv, p,
                         preferred_element_type=jnp.float32) + b_ref[...]


def _combine(acc2, den2, bias2):
    return pl.pallas_call(
        _comb_body,
        grid=(GRID_NP,),
        in_specs=[
            pl.BlockSpec((2, BLK, D), lambda i: (0, i, 0)),
            pl.BlockSpec((2, BLK), lambda i: (0, i)),
            pl.BlockSpec((1, D), lambda i: (0, 0)),
        ],
        out_specs=pl.BlockSpec((BLK, D), lambda i: (i, 0)),
        out_shape=jax.ShapeDtypeStruct((N_PAD, D), jnp.float32),
    )(acc2, den2, bias2)


def kernel(x, edge_index, W, att_src, att_dst, bias):
    ei = edge_index.astype(jnp.int32)
    loop = jnp.arange(N, dtype=jnp.int32)
    pad = jnp.zeros((E_PAD - E_TOT,), jnp.int32)
    src = jnp.concatenate([ei[0], loop, pad])
    dst = jnp.concatenate([ei[1], loop, pad])

    att = jnp.stack([att_src, att_dst])
    h, a2 = _project(x, W, att)
    acc2, den2 = _sc_aggregate(src, dst, h, a2[0], a2[1])
    return _combine(acc2, den2, bias.reshape(1, D))[:N]
